# baseline (speedup 1.0000x reference)
import functools

import jax
import jax.numpy as jnp
from jax.experimental import pallas as pl

N = 100000
E = 1600000
R = 400000


def _out_body(p_ref, rf_ref, w_ref, o_ref):
    o_ref[...] = p_ref[...] + rf_ref[...] @ w_ref[...]


def kernel(edge_index, edge_attr, route_idx, route_feats,
           Wa0, ba0, Wn0, bn0, Wa1, ba1, Wn1, bn1, Wo, bo):
    src = edge_index[0]
    dst = edge_index[1]
    deg = jnp.zeros((N,), jnp.float32).at[dst].add(1.0)
    emb = (deg / float(N))[:, None]
    for (Wa, ba, Wn, bn) in ((Wa0, ba0, Wn0, bn0), (Wa1, ba1, Wn1, bn1)):
        nb = jnp.concatenate([emb[src], edge_attr], axis=1)
        proj = jax.nn.relu(nb @ Wa + ba)
        agg = jax.ops.segment_max(proj, dst, num_segments=N)
        agg = jnp.where(jnp.isfinite(agg), agg, 0.0)
        emb = jax.nn.relu(jnp.concatenate([agg, emb], axis=1) @ Wn + bn)

    p = (emb @ Wo[:32])[:, 0] + bo[0]
    pg = p[route_idx][:, None]
    BR = 8000
    out = pl.pallas_call(
        _out_body,
        grid=(R // BR,),
        in_specs=[
            pl.BlockSpec((BR, 1), lambda i: (i, 0)),
            pl.BlockSpec((BR, 8), lambda i: (i, 0)),
            pl.BlockSpec((8, 1), lambda i: (0, 0)),
        ],
        out_specs=pl.BlockSpec((BR, 1), lambda i: (i, 0)),
        out_shape=jax.ShapeDtypeStruct((R, 1), jnp.float32),
    )
    rf = jnp.pad(route_feats, ((0, 0), (0, 2)))
    wo = jnp.pad(Wo[32:], ((0, 2), (0, 0)))
    logits = out(pg, rf, wo)[:, 0]
    return logits


# speedup vs baseline: 1.7046x; 1.7046x over previous
import functools

import jax
import jax.numpy as jnp
from jax import lax
from jax.experimental import pallas as pl
from jax.experimental.pallas import tpu as pltpu
from jax.experimental.pallas import tpu_sc as plsc

N = 100000
N2 = 102400
E = 1600000
R = 400000
F = 32
NW = 32
EC = E // NW
WIN = 2000
GWIN = 400
RW = 2000
BN = 4096
BE = 16000
BR = 8000

_IOTA = lambda: lax.iota(jnp.int32, 16)


def _sc_mesh():
    return plsc.VectorSubcoreMesh(core_axis_name="c", subcore_axis_name="s")


def _wid():
    return lax.axis_index("s") * 2 + lax.axis_index("c")


def _zero_f32(ref, nwords):
    z = jnp.zeros((16,), jnp.float32)

    def body(i, _):
        ref[pl.ds(i * 16, 16)] = z
        return ()

    lax.fori_loop(0, nwords // 16, body, ())


@functools.partial(
    pl.kernel,
    out_type=jax.ShapeDtypeStruct((NW, N2), jnp.float32),
    mesh=_sc_mesh(),
    compiler_params=pltpu.CompilerParams(needs_layout_passes=False, use_tc_tiling_on_sc=False),
    scratch_types=[
        pltpu.VMEM((N2,), jnp.float32),
        pltpu.VMEM((WIN,), jnp.int32),
    ],
)
def _deg_kernel(dst_hbm, out_hbm, acc, dstw):
    w = _wid()
    _zero_f32(acc, N2)
    base0 = w * EC

    def wbody(j, _):
        pltpu.sync_copy(dst_hbm.at[pl.ds(base0 + j * WIN, WIN)], dstw)

        ones = jnp.ones((16,), jnp.float32)
        lanes = _IOTA()

        def vbody(v, _):
            d = dstw[pl.ds(v * 16, 16)]
            for l in range(16):
                cur = plsc.load_gather(acc, [d])
                plsc.store_scatter(acc, [d], cur + ones, mask=lanes == l)
            return ()

        lax.fori_loop(0, WIN // 16, vbody, ())
        return ()

    lax.fori_loop(0, EC // WIN, wbody, ())
    pltpu.sync_copy(acc, out_hbm.at[w])


@functools.partial(
    pl.kernel,
    out_type=jax.ShapeDtypeStruct((F, E), jnp.float32),
    mesh=_sc_mesh(),
    compiler_params=pltpu.CompilerParams(needs_layout_passes=False, use_tc_tiling_on_sc=False),
    scratch_types=[
        pltpu.VMEM((GWIN,), jnp.int32),
        pltpu.VMEM((GWIN, F), jnp.float32),
        pltpu.VMEM((F, GWIN), jnp.float32),
        pltpu.SemaphoreType.DMA,
    ],
)
def _gt_kernel(g_hbm, src_hbm, out_hbm, idxv, rows, tbuf, sem):
    w = _wid()
    base0 = w * EC

    def wbody(j, _):
        b = base0 + j * GWIN
        pltpu.sync_copy(src_hbm.at[pl.ds(b, GWIN)], idxv)
        pltpu.async_copy(g_hbm.at[idxv], rows, sem).wait()
        for f in range(F):
            col = jnp.full((16,), f, jnp.int32)

            def rbody(jj, _):
                ridx = jj * 16 + _IOTA()
                vals = plsc.load_gather(rows, [ridx, col])
                tbuf[f, pl.ds(jj * 16, 16)] = vals
                return ()

            lax.fori_loop(0, GWIN // 16, rbody, ())
        pltpu.sync_copy(tbuf, out_hbm.at[:, pl.ds(b, GWIN)])
        return ()

    lax.fori_loop(0, EC // GWIN, wbody, ())


@functools.partial(
    pl.kernel,
    out_type=jax.ShapeDtypeStruct((F, N2), jnp.float32),
    mesh=_sc_mesh(),
    compiler_params=pltpu.CompilerParams(needs_layout_passes=False, use_tc_tiling_on_sc=False),
    scratch_types=[
        pltpu.VMEM((N2,), jnp.float32),
        pltpu.VMEM((WIN,), jnp.int32),
        pltpu.VMEM((WIN,), jnp.float32),
        pltpu.VMEM((WIN,), jnp.float32),
    ],
)
def _rmw_kernel(dst_hbm, gsT_hbm, hT_hbm, out_hbm, acc, dstw, gw, hw):
    f = _wid()
    _zero_f32(acc, N2)

    def wbody(j, _):
        b = j * WIN
        pltpu.sync_copy(dst_hbm.at[pl.ds(b, WIN)], dstw)
        pltpu.sync_copy(gsT_hbm.at[f, pl.ds(b, WIN)], gw)
        pltpu.sync_copy(hT_hbm.at[f, pl.ds(b, WIN)], hw)

        def first_pass():
            def vbody(v, anyfail):
                sl = pl.ds(v * 16, 16)
                d = dstw[sl]
                tgt = gw[sl] + hw[sl]
                cur = plsc.load_gather(acc, [d])
                plsc.store_scatter(acc, [d], jnp.maximum(cur, tgt))
                chk = plsc.load_gather(acc, [d])
                return anyfail | (chk < tgt)

            return lax.fori_loop(0, WIN // 16, vbody,
                                 jnp.zeros((16,), jnp.bool_))

        def redo_pass(_):
            def vbody(v, anyfail):
                sl = pl.ds(v * 16, 16)
                d = dstw[sl]
                tgt = gw[sl] + hw[sl]
                cur = plsc.load_gather(acc, [d])
                plsc.store_scatter(acc, [d], tgt, mask=cur < tgt)
                chk = plsc.load_gather(acc, [d])
                return anyfail | (chk < tgt)

            return lax.fori_loop(0, WIN // 16, vbody,
                                 jnp.zeros((16,), jnp.bool_))

        af = first_pass()

        def cond(afc):
            return plsc.all_reduce_population_count(afc)[0] > 0

        lax.while_loop(cond, redo_pass, af)
        return ()

    lax.fori_loop(0, E // WIN, wbody, ())
    pltpu.sync_copy(acc, out_hbm.at[f])


@functools.partial(
    pl.kernel,
    out_type=jax.ShapeDtypeStruct((R,), jnp.float32),
    mesh=_sc_mesh(),
    compiler_params=pltpu.CompilerParams(needs_layout_passes=False, use_tc_tiling_on_sc=False),
    scratch_types=[
        pltpu.VMEM((RW,), jnp.int32),
        pltpu.VMEM((RW, 16), jnp.float32),
        pltpu.VMEM((RW,), jnp.float32),
        pltpu.SemaphoreType.DMA,
    ],
)
def _route_kernel(pw_hbm, ridx_hbm, out_hbm, idxv, rows, prw, sem):
    w = _wid()
    nwin = R // RW
    col0 = jnp.zeros((16,), jnp.int32)

    def wbody(t, _):
        j = w + t * NW

        @pl.when(j < nwin)
        def _():
            b = j * RW
            pltpu.sync_copy(ridx_hbm.at[pl.ds(b, RW)], idxv)
            pltpu.async_copy(pw_hbm.at[idxv], rows, sem).wait()

            def vbody(v, _):
                ridx = v * 16 + _IOTA()
                vals = plsc.load_gather(rows, [ridx, col0])
                prw[pl.ds(v * 16, 16)] = vals
                return ()

            lax.fori_loop(0, RW // 16, vbody, ())
            pltpu.sync_copy(prw, out_hbm.at[pl.ds(b, RW)])

        return ()

    lax.fori_loop(0, (nwin + NW - 1) // NW, wbody, ())


def _prep0_body(degp_ref, wa0_ref, emb0_ref, g0_ref):
    d = jnp.sum(degp_ref[...], axis=0, keepdims=True)
    emb0 = d.T / float(N)
    emb0_ref[...] = emb0
    g0_ref[...] = emb0 * wa0_ref[0:1, :]


def _h_body(ea_ref, wa0_ref, wa1_ref, ba0_ref, ba1_ref, h0_ref, h1_ref):
    ea = ea_ref[...]
    dn = (((0,), (1,)), ((), ()))
    h0_ref[...] = lax.dot_general(wa0_ref[...][1:4], ea, dn) + ba0_ref[...].T
    h1_ref[...] = lax.dot_general(wa1_ref[...][32:35], ea, dn) + ba1_ref[...].T


def _comb0_body(aggT_ref, emb0_ref, wn0_ref, wa1_ref, bn0_ref,
                emb1_ref, g1_ref):
    wn0 = wn0_ref[...]
    dn = (((0,), (0,)), ((), ()))
    agg_wn = lax.dot_general(aggT_ref[...], wn0[:F], dn)
    emb1 = jax.nn.relu(agg_wn + emb0_ref[...] * wn0[F:F + 1] + bn0_ref[...])
    emb1_ref[...] = emb1
    g1_ref[...] = emb1 @ wa1_ref[...][:F]


def _comb1_body(aggT_ref, emb1_ref, wn1_ref, wo_ref, bn1_ref, bo_ref,
                pw_ref):
    wn1 = wn1_ref[...]
    dn = (((0,), (0,)), ((), ()))
    agg_wn = lax.dot_general(aggT_ref[...], wn1[:F], dn)
    emb2 = jax.nn.relu(agg_wn + emb1_ref[...] @ wn1[F:] + bn1_ref[...])
    p = emb2 @ wo_ref[...][:F] + bo_ref[0, 0]
    pw_ref[...] = jnp.broadcast_to(p, (p.shape[0], 16))


def _out_body(pr_ref, rf_ref, w_ref, o_ref):
    o_ref[...] = pr_ref[...] + rf_ref[...] @ w_ref[...]


def kernel(edge_index, edge_attr, route_idx, route_feats,
           Wa0, ba0, Wn0, bn0, Wa1, ba1, Wn1, bn1, Wo, bo):
    src = edge_index[0]
    dst = edge_index[1]

    degp = _deg_kernel(dst)
    emb0, g0 = pl.pallas_call(
        _prep0_body,
        grid=(N2 // BN,),
        in_specs=[
            pl.BlockSpec((NW, BN), lambda i: (0, i)),
            pl.BlockSpec((4, F), lambda i: (0, 0)),
        ],
        out_specs=[
            pl.BlockSpec((BN, 1), lambda i: (i, 0)),
            pl.BlockSpec((BN, F), lambda i: (i, 0)),
        ],
        out_shape=[
            jax.ShapeDtypeStruct((N2, 1), jnp.float32),
            jax.ShapeDtypeStruct((N2, F), jnp.float32),
        ],
    )(degp, Wa0)

    h0T, h1T = pl.pallas_call(
        _h_body,
        grid=(E // BE,),
        in_specs=[
            pl.BlockSpec((BE, 3), lambda i: (i, 0)),
            pl.BlockSpec((4, F), lambda i: (0, 0)),
            pl.BlockSpec((35, F), lambda i: (0, 0)),
            pl.BlockSpec((1, F), lambda i: (0, 0)),
            pl.BlockSpec((1, F), lambda i: (0, 0)),
        ],
        out_specs=[
            pl.BlockSpec((F, BE), lambda i: (0, i)),
            pl.BlockSpec((F, BE), lambda i: (0, i)),
        ],
        out_shape=[
            jax.ShapeDtypeStruct((F, E), jnp.float32),
            jax.ShapeDtypeStruct((F, E), jnp.float32),
        ],
    )(edge_attr, Wa0, Wa1, ba0.reshape(1, F), ba1.reshape(1, F))

    gs0T = _gt_kernel(g0, src)
    agg0T = _rmw_kernel(dst, gs0T, h0T)
    emb1, g1 = pl.pallas_call(
        _comb0_body,
        grid=(N2 // BN,),
        in_specs=[
            pl.BlockSpec((F, BN), lambda i: (0, i)),
            pl.BlockSpec((BN, 1), lambda i: (i, 0)),
            pl.BlockSpec((F + 1, F), lambda i: (0, 0)),
            pl.BlockSpec((35, F), lambda i: (0, 0)),
            pl.BlockSpec((1, F), lambda i: (0, 0)),
        ],
        out_specs=[
            pl.BlockSpec((BN, F), lambda i: (i, 0)),
            pl.BlockSpec((BN, F), lambda i: (i, 0)),
        ],
        out_shape=[
            jax.ShapeDtypeStruct((N2, F), jnp.float32),
            jax.ShapeDtypeStruct((N2, F), jnp.float32),
        ],
    )(agg0T, emb0, Wn0, Wa1, bn0.reshape(1, F))

    gs1T = _gt_kernel(g1, src)
    agg1T = _rmw_kernel(dst, gs1T, h1T)
    pw = pl.pallas_call(
        _comb1_body,
        grid=(N2 // BN,),
        in_specs=[
            pl.BlockSpec((F, BN), lambda i: (0, i)),
            pl.BlockSpec((BN, F), lambda i: (i, 0)),
            pl.BlockSpec((2 * F, F), lambda i: (0, 0)),
            pl.BlockSpec((F + 6, 1), lambda i: (0, 0)),
            pl.BlockSpec((1, F), lambda i: (0, 0)),
            pl.BlockSpec((1, 1), lambda i: (0, 0)),
        ],
        out_specs=pl.BlockSpec((BN, 16), lambda i: (i, 0)),
        out_shape=jax.ShapeDtypeStruct((N2, 16), jnp.float32),
    )(agg1T, emb1, Wn1, Wo, bn1.reshape(1, F), bo.reshape(1, 1))

    pr = _route_kernel(pw, route_idx)
    logits = pl.pallas_call(
        _out_body,
        grid=(R // BR,),
        in_specs=[
            pl.BlockSpec((BR, 1), lambda i: (i, 0)),
            pl.BlockSpec((BR, 8), lambda i: (i, 0)),
            pl.BlockSpec((8, 1), lambda i: (0, 0)),
        ],
        out_specs=pl.BlockSpec((BR, 1), lambda i: (i, 0)),
        out_shape=jax.ShapeDtypeStruct((R, 1), jnp.float32),
    )(
        pr.reshape(R, 1),
        jnp.pad(route_feats, ((0, 0), (0, 2))),
        jnp.pad(Wo[F:], ((0, 2), (0, 0))),
    )[:, 0]
    return logits
